# Initial kernel scaffold; baseline (speedup 1.0000x reference)
import jax
import jax.numpy as jnp
from jax.experimental import pallas as pl
from jax.experimental.pallas import tpu as pltpu
from jax.experimental.pallas import tpu_sc as plsc

_WINDOW = 1024


def _gather_kernel(num_indices, dim, dtype):
    vector_mesh = plsc.VectorSubcoreMesh(
        core_axis_name="core", subcore_axis_name="subcore"
    )

    @pl.kernel(
        out_type=jax.ShapeDtypeStruct((num_indices, dim), dtype),
        mesh=vector_mesh,
    )
    def kernel_fn(table_hbm, idx_hbm, out_hbm):
        def body(idx_vmem, out_vmem):
            pltpu.sync_copy(table_hbm.at[idx_vmem.at[0]], out_vmem)

        pltpu.emit_pipeline(
            body,
            grid=(num_indices // _WINDOW,),
            in_specs=[pl.BlockSpec((1, _WINDOW), index_map=lambda i: (0, i))],
            out_specs=[pl.BlockSpec((_WINDOW, dim), index_map=lambda i: (i, 0))],
            core_axis_name=("core", "subcore"),
            dimension_semantics=(pltpu.PARALLEL,),
        )(idx_hbm, out_hbm)

    return kernel_fn


def kernel(token_ids, weight):
    batch, hist = token_ids.shape
    num_indices = batch * hist
    dim = weight.shape[1]
    flat_idx = token_ids.reshape(1, num_indices).astype(jnp.int32)
    out = _gather_kernel(num_indices, dim, weight.dtype)(weight, flat_idx)
    return out.reshape(batch, hist, dim)


# speedup vs baseline: 1.0955x; 1.0955x over previous
import functools

import jax
import jax.numpy as jnp
from jax import lax
from jax.experimental import pallas as pl
from jax.experimental.pallas import tpu as pltpu
from jax.experimental.pallas import tpu_sc as plsc

_NUM_WORKERS = 32
_CHUNK = 1024


def _make_gather(num_indices, dim, dtype):
    b_per_w = num_indices // _NUM_WORKERS
    n_chunks = b_per_w // _CHUNK
    mesh = plsc.VectorSubcoreMesh(core_axis_name="c", subcore_axis_name="s")

    @functools.partial(
        pl.kernel,
        mesh=mesh,
        compiler_params=pltpu.CompilerParams(use_tc_tiling_on_sc=False),
        out_type=jax.ShapeDtypeStruct((num_indices, dim), dtype),
        scratch_types=[
            pltpu.VMEM((_CHUNK,), jnp.int32),
            pltpu.VMEM((_CHUNK, dim), dtype),
            pltpu.SemaphoreType.DMA,
        ],
    )
    def gather_kernel(table_hbm, idx_hbm, out_hbm, idx_v, rows_v, sem):
        wid = lax.axis_index("s") * 2 + lax.axis_index("c")
        wbase = wid * b_per_w

        @pl.loop(0, n_chunks)
        def _(c):
            base = wbase + c * _CHUNK
            pltpu.sync_copy(idx_hbm.at[pl.ds(base, _CHUNK)], idx_v)
            pltpu.async_copy(table_hbm.at[idx_v], rows_v, sem).wait()
            pltpu.sync_copy(rows_v, out_hbm.at[pl.ds(base, _CHUNK)])

    return gather_kernel


def kernel(token_ids, weight):
    batch, hist = token_ids.shape
    num_indices = batch * hist
    dim = weight.shape[1]
    flat_idx = token_ids.reshape(num_indices).astype(jnp.int32)
    out = _make_gather(num_indices, dim, weight.dtype)(weight, flat_idx)
    return out.reshape(batch, hist, dim)


# speedup vs baseline: 1.1133x; 1.0162x over previous
import functools

import jax
import jax.numpy as jnp
from jax import lax
from jax.experimental import pallas as pl
from jax.experimental.pallas import tpu as pltpu
from jax.experimental.pallas import tpu_sc as plsc

_NUM_WORKERS = 32
_CHUNK = 1024
_NBUF = 3


def _make_gather(num_indices, dim, dtype):
    b_per_w = num_indices // _NUM_WORKERS
    n_chunks = b_per_w // _CHUNK
    mesh = plsc.VectorSubcoreMesh(core_axis_name="c", subcore_axis_name="s")

    @functools.partial(
        pl.kernel,
        mesh=mesh,
        compiler_params=pltpu.CompilerParams(use_tc_tiling_on_sc=False),
        out_type=jax.ShapeDtypeStruct((num_indices, dim), dtype),
        scratch_types=[pltpu.VMEM((b_per_w,), jnp.int32)]
        + [pltpu.VMEM((_CHUNK, dim), dtype) for _ in range(_NBUF)]
        + [pltpu.SemaphoreType.DMA for _ in range(2 * _NBUF + 1)],
    )
    def gather_kernel(table_hbm, idx_hbm, out_hbm, *scratch):
        idx_v = scratch[0]
        rbufs = scratch[1 : 1 + _NBUF]
        gsems = scratch[1 + _NBUF : 1 + 2 * _NBUF]
        wsems = scratch[1 + 2 * _NBUF : 1 + 3 * _NBUF]
        isem = scratch[1 + 3 * _NBUF]

        wid = lax.axis_index("s") * 2 + lax.axis_index("c")
        wbase = wid * b_per_w

        def gather(c, b):
            return pltpu.async_copy(
                table_hbm.at[idx_v.at[pl.ds(c * _CHUNK, _CHUNK)]],
                rbufs[b],
                gsems[b],
            )

        def writeback(c, b):
            return pltpu.async_copy(
                rbufs[b],
                out_hbm.at[pl.ds(wbase + c * _CHUNK, _CHUNK)],
                wsems[b],
            )

        pltpu.async_copy(
            idx_hbm.at[pl.ds(wbase, b_per_w)], idx_v, isem
        ).wait()

        g_h = [None] * n_chunks
        w_h = [None] * n_chunks
        for c in range(min(_NBUF, n_chunks)):
            g_h[c] = gather(c, c % _NBUF)
        for c in range(n_chunks):
            b = c % _NBUF
            g_h[c].wait()
            w_h[c] = writeback(c, b)
            prev = c - 1
            if 0 <= prev and prev + _NBUF < n_chunks:
                w_h[prev].wait()
                g_h[prev + _NBUF] = gather(prev + _NBUF, prev % _NBUF)
        for c in range(max(0, n_chunks - _NBUF), n_chunks):
            w_h[c].wait()

    return gather_kernel


def kernel(token_ids, weight):
    batch, hist = token_ids.shape
    num_indices = batch * hist
    dim = weight.shape[1]
    flat_idx = token_ids.reshape(num_indices).astype(jnp.int32)
    out = _make_gather(num_indices, dim, weight.dtype)(weight, flat_idx)
    return out.reshape(batch, hist, dim)


# speedup vs baseline: 1.7499x; 1.5719x over previous
import functools

import jax
import jax.numpy as jnp
from jax import lax
from jax.experimental import pallas as pl
from jax.experimental.pallas import tpu as pltpu
from jax.experimental.pallas import tpu_sc as plsc

_NUM_WORKERS = 32
_CHUNK = 1024
_NBUF = 3


def _make_gather(num_indices, dim, dtype):
    b_per_w = num_indices // _NUM_WORKERS
    n_chunks = b_per_w // _CHUNK
    nq = 128 // dim
    quarter = num_indices // nq
    q_per_w = b_per_w // nq
    q_chunk = _CHUNK // nq
    rows128 = _CHUNK * dim // 128
    mesh = plsc.VectorSubcoreMesh(core_axis_name="c", subcore_axis_name="s")

    @functools.partial(
        pl.kernel,
        mesh=mesh,
        compiler_params=pltpu.CompilerParams(use_tc_tiling_on_sc=False),
        out_type=jax.ShapeDtypeStruct((num_indices * dim // 128, 128), dtype),
        scratch_types=[pltpu.VMEM((b_per_w,), jnp.int32)]
        + [pltpu.VMEM((_CHUNK, dim), dtype) for _ in range(_NBUF)]
        + [pltpu.SemaphoreType.DMA for _ in range(2 * _NBUF + 1)],
    )
    def gather_kernel(table_hbm, idxp_hbm, out128_hbm, *scratch):
        idx_v = scratch[0]
        rbufs = scratch[1 : 1 + _NBUF]
        gsems = scratch[1 + _NBUF : 1 + 2 * _NBUF]
        wsems = scratch[1 + 2 * _NBUF : 1 + 3 * _NBUF]
        isem = scratch[1 + 3 * _NBUF]

        wid = lax.axis_index("s") * 2 + lax.axis_index("c")
        wbase4 = wid * q_per_w
        wbase128 = wid * (b_per_w * dim // 128)

        def gather(c, b):
            return [
                pltpu.async_copy(
                    table_hbm.at[
                        idx_v.at[pl.ds(q * q_per_w + c * q_chunk, q_chunk)]
                    ],
                    rbufs[b].at[pl.ds(q * q_chunk, q_chunk)],
                    gsems[b],
                )
                for q in range(nq)
            ]

        def writeback(c, b):
            return [
                pltpu.async_copy(
                    rbufs[b].at[pl.ds(q * q_chunk, q_chunk)],
                    out128_hbm.at[
                        pl.ds(wbase128 + c * rows128, rows128),
                        pl.ds(q * dim, dim),
                    ],
                    wsems[b],
                )
                for q in range(nq)
            ]

        ih = [
            pltpu.async_copy(
                idxp_hbm.at[pl.ds(q * quarter + wbase4, q_per_w)],
                idx_v.at[pl.ds(q * q_per_w, q_per_w)],
                isem,
            )
            for q in range(nq)
        ]
        for h in ih:
            h.wait()

        g_h = [None] * n_chunks
        w_h = [None] * n_chunks
        for c in range(min(_NBUF, n_chunks)):
            g_h[c] = gather(c, c % _NBUF)
        for c in range(n_chunks):
            b = c % _NBUF
            for h in g_h[c]:
                h.wait()
            w_h[c] = writeback(c, b)
            prev = c - 1
            if 0 <= prev and prev + _NBUF < n_chunks:
                for h in w_h[prev]:
                    h.wait()
                g_h[prev + _NBUF] = gather(prev + _NBUF, prev % _NBUF)
        for c in range(max(0, n_chunks - _NBUF), n_chunks):
            for h in w_h[c]:
                h.wait()

    return gather_kernel


def kernel(token_ids, weight):
    batch, hist = token_ids.shape
    num_indices = batch * hist
    num_rows, dim = weight.shape
    nq = 128 // dim
    flat_idx = token_ids.reshape(num_indices).astype(jnp.int32)
    idx_perm = flat_idx.reshape(num_indices // nq, nq).T.reshape(num_indices)
    out128 = _make_gather(num_indices, dim, weight.dtype)(weight, idx_perm)
    return out128.reshape(batch, hist, dim)


# speedup vs baseline: 1.7542x; 1.0024x over previous
import functools

import jax
import jax.numpy as jnp
from jax import lax
from jax.experimental import pallas as pl
from jax.experimental.pallas import tpu as pltpu
from jax.experimental.pallas import tpu_sc as plsc

_NUM_WORKERS = 32
_CHUNK = 512
_NBUF = 3


def _make_gather(num_indices, dim, dtype):
    b_per_w = num_indices // _NUM_WORKERS
    n_chunks = b_per_w // _CHUNK
    nq = 128 // dim
    quarter = num_indices // nq
    q_per_w = b_per_w // nq
    q_chunk = _CHUNK // nq
    rows128 = _CHUNK * dim // 128
    mesh = plsc.VectorSubcoreMesh(core_axis_name="c", subcore_axis_name="s")

    @functools.partial(
        pl.kernel,
        mesh=mesh,
        compiler_params=pltpu.CompilerParams(use_tc_tiling_on_sc=False),
        out_type=jax.ShapeDtypeStruct((num_indices * dim // 128, 128), dtype),
        scratch_types=[pltpu.VMEM((b_per_w,), jnp.int32)]
        + [pltpu.VMEM((_CHUNK, dim), dtype) for _ in range(_NBUF)]
        + [pltpu.SemaphoreType.DMA for _ in range(2 * _NBUF + 1)],
    )
    def gather_kernel(table_hbm, idxp_hbm, out128_hbm, *scratch):
        idx_v = scratch[0]
        rbufs = scratch[1 : 1 + _NBUF]
        gsems = scratch[1 + _NBUF : 1 + 2 * _NBUF]
        wsems = scratch[1 + 2 * _NBUF : 1 + 3 * _NBUF]
        isem = scratch[1 + 3 * _NBUF]

        wid = lax.axis_index("s") * 2 + lax.axis_index("c")
        wbase4 = wid * q_per_w
        wbase128 = wid * (b_per_w * dim // 128)

        def gather(c, b):
            return [
                pltpu.async_copy(
                    table_hbm.at[
                        idx_v.at[pl.ds(q * q_per_w + c * q_chunk, q_chunk)]
                    ],
                    rbufs[b].at[pl.ds(q * q_chunk, q_chunk)],
                    gsems[b],
                )
                for q in range(nq)
            ]

        def writeback(c, b):
            return [
                pltpu.async_copy(
                    rbufs[b].at[pl.ds(q * q_chunk, q_chunk)],
                    out128_hbm.at[
                        pl.ds(wbase128 + c * rows128, rows128),
                        pl.ds(q * dim, dim),
                    ],
                    wsems[b],
                )
                for q in range(nq)
            ]

        ih = [
            pltpu.async_copy(
                idxp_hbm.at[pl.ds(q * quarter + wbase4, q_per_w)],
                idx_v.at[pl.ds(q * q_per_w, q_per_w)],
                isem,
            )
            for q in range(nq)
        ]
        for h in ih:
            h.wait()

        g_h = [None] * n_chunks
        w_h = [None] * n_chunks
        for c in range(min(_NBUF, n_chunks)):
            g_h[c] = gather(c, c % _NBUF)
        for c in range(n_chunks):
            b = c % _NBUF
            for h in g_h[c]:
                h.wait()
            w_h[c] = writeback(c, b)
            prev = c - 1
            if 0 <= prev and prev + _NBUF < n_chunks:
                for h in w_h[prev]:
                    h.wait()
                g_h[prev + _NBUF] = gather(prev + _NBUF, prev % _NBUF)
        for c in range(max(0, n_chunks - _NBUF), n_chunks):
            for h in w_h[c]:
                h.wait()

    return gather_kernel


def kernel(token_ids, weight):
    batch, hist = token_ids.shape
    num_indices = batch * hist
    num_rows, dim = weight.shape
    nq = 128 // dim
    flat_idx = token_ids.reshape(num_indices).astype(jnp.int32)
    idx_perm = flat_idx.reshape(num_indices // nq, nq).T.reshape(num_indices)
    out128 = _make_gather(num_indices, dim, weight.dtype)(weight, idx_perm)
    return out128.reshape(batch, hist, dim)
